# Initial kernel scaffold; baseline (speedup 1.0000x reference)
import jax
import jax.numpy as jnp
from jax.experimental import pallas as pl


def kernel(x, edge_index, W1, b1, g1, be1, W2, b2, g2, be2, W3, b3):
    raise NotImplementedError("write your pallas kernel here")


# speedup vs baseline: 19.3227x; 19.3227x over previous
import functools
import math

import jax
import jax.numpy as jnp
from jax import lax
from jax.experimental import pallas as pl
from jax.experimental.pallas import tpu as pltpu
from jax.experimental.pallas import tpu_sc as plsc

N = 10000
D = 128
E = 320000
EPS = 1e-5
INVC = 1.0 / math.sqrt(1.0 + EPS)

NC, NS = 2, 16
NW = NC * NS
CHUNK = 128
NCHUNK = 80
EP = NW * NCHUNK * CHUNK
NPAD = 10240
RPS = NPAD // NS

def _deg_body(dst_hbm, zdeg_hbm, cnt_hbm, idx_v, ones_v, acc_sh):
    c = lax.axis_index("c")
    s = lax.axis_index("s")
    pltpu.sync_copy(dst_hbm.at[c, s], idx_v)
    for i in range(CHUNK // 16):
        ones_v[pl.ds(i * 16, 16)] = jnp.full((16,), 1.0, jnp.float32)
    pltpu.sync_copy(zdeg_hbm, acc_sh.at[pl.ds(s * RPS, RPS)])
    plsc.subcore_barrier()

    def chunk(j, carry):
        pltpu.sync_copy(ones_v, acc_sh.at[idx_v.at[j]], add=True)
        return carry

    lax.fori_loop(0, NCHUNK, chunk, 0)
    plsc.subcore_barrier()
    pltpu.sync_copy(acc_sh.at[pl.ds(s * RPS, RPS)], cnt_hbm.at[c, pl.ds(s * RPS, RPS)])


def _scatter_body(hp_hbm, src_hbm, dst_hbm, zrows_hbm, out_hbm,
                  src_v, dst_v, rows_v, gsem, ssem, acc_sh):
    c = lax.axis_index("c")
    s = lax.axis_index("s")
    pltpu.sync_copy(src_hbm.at[c, s], src_v)
    pltpu.sync_copy(dst_hbm.at[c, s], dst_v)
    pltpu.sync_copy(zrows_hbm, acc_sh.at[pl.ds(s * RPS, RPS)])
    plsc.subcore_barrier()

    def chunk(j, carry):
        pltpu.async_copy(hp_hbm.at[src_v.at[j]], rows_v, gsem).wait()
        pltpu.async_copy(rows_v, acc_sh.at[dst_v.at[j]], ssem,
                         add=True).wait()
        return carry

    lax.fori_loop(0, NCHUNK, chunk, 0)
    plsc.subcore_barrier()
    pltpu.sync_copy(acc_sh.at[pl.ds(s * RPS, RPS)],
                    out_hbm.at[c, pl.ds(s * RPS, RPS)])


@functools.cache
def _sc_kernels():
    mesh = plsc.VectorSubcoreMesh(core_axis_name="c", subcore_axis_name="s",
                                  num_cores=NC, num_subcores=NS)
    deg = pl.kernel(
        _deg_body,
        out_type=jax.ShapeDtypeStruct((NC, NPAD), jnp.float32),
        mesh=mesh,
        scratch_types=[
            pltpu.VMEM((NCHUNK, CHUNK), jnp.int32),
            pltpu.VMEM((CHUNK,), jnp.float32),
            pltpu.VMEM_SHARED((NPAD,), jnp.float32),
        ],
    )
    scat = pl.kernel(
        _scatter_body,
        out_type=jax.ShapeDtypeStruct((NC, NPAD, D), jnp.float32),
        mesh=mesh,
        scratch_types=[
            pltpu.VMEM((NCHUNK, CHUNK), jnp.int32),
            pltpu.VMEM((NCHUNK, CHUNK), jnp.int32),
            pltpu.VMEM((CHUNK, D), jnp.float32),
            pltpu.SemaphoreType.DMA,
            pltpu.SemaphoreType.DMA,
            pltpu.VMEM_SHARED((NPAD, D), jnp.float32),
        ],
    )
    return deg, scat


BR = 512
GRID = NPAD // BR


def _dis_from_cnt(cnt_ref):
    cnt = cnt_ref[0, :] + cnt_ref[1, :]
    return lax.rsqrt(cnt + 1.0)[:, None]


def _mm1_body(cnt_ref, x_ref, w_ref, o_ref):
    h = jnp.dot(x_ref[...], w_ref[...], preferred_element_type=jnp.float32)
    o_ref[...] = h * _dis_from_cnt(cnt_ref)


def _mid_body(cnt_ref, s2_ref, hp_ref, b_ref, g_ref, be_ref, w_ref, o_ref):
    dis = _dis_from_cnt(cnt_ref)
    conv = dis * (s2_ref[0] + s2_ref[1] + hp_ref[...]) + b_ref[...]
    a = jnp.maximum(g_ref[...] * (conv * INVC) + be_ref[...], 0.0)
    o_ref[...] = jnp.dot(a, w_ref[...], preferred_element_type=jnp.float32) * dis


def _final_body(cnt_ref, s2_ref, hp_ref, b_ref, o_ref):
    dis = _dis_from_cnt(cnt_ref)
    o_ref[...] = dis * (s2_ref[0] + s2_ref[1] + hp_ref[...]) + b_ref[...]


_cnt_spec = pl.BlockSpec((2, BR), lambda i: (0, i))
_row_spec = pl.BlockSpec((BR, D), lambda i: (i, 0))
_s2_spec = pl.BlockSpec((2, BR, D), lambda i: (0, i, 0))
_vec_spec = pl.BlockSpec((1, D), lambda i: (0, 0))
_w_spec = pl.BlockSpec((D, D), lambda i: (0, 0))
_out_sds = jax.ShapeDtypeStruct((NPAD, D), jnp.float32)

_mm1 = pl.pallas_call(
    _mm1_body, grid=(GRID,),
    in_specs=[_cnt_spec, _row_spec, _w_spec],
    out_specs=_row_spec, out_shape=_out_sds)

_mid = pl.pallas_call(
    _mid_body, grid=(GRID,),
    in_specs=[_cnt_spec, _s2_spec, _row_spec, _vec_spec, _vec_spec, _vec_spec,
              _w_spec],
    out_specs=_row_spec, out_shape=_out_sds)

_final = pl.pallas_call(
    _final_body, grid=(GRID,),
    in_specs=[_cnt_spec, _s2_spec, _row_spec, _vec_spec],
    out_specs=_row_spec, out_shape=_out_sds)


def kernel(x, edge_index, W1, b1, g1, be1, W2, b2, g2, be2, W3, b3):
    pad_idx = (N + (jnp.arange(EP - E, dtype=jnp.int32) % (NPAD - N)))
    src_p = jnp.concatenate([edge_index[0], pad_idx]).reshape(NC, NS, NCHUNK, CHUNK)
    dst_p = jnp.concatenate([edge_index[1], pad_idx]).reshape(NC, NS, NCHUNK, CHUNK)
    zdeg = jnp.zeros((RPS,), jnp.float32)
    zrows = jnp.zeros((RPS, D), jnp.float32)
    x_p = jnp.pad(x, ((0, NPAD - N), (0, 0)))
    b1r, b2r, b3r = b1.reshape(1, D), b2.reshape(1, D), b3.reshape(1, D)
    g1r, g2r = g1.reshape(1, D), g2.reshape(1, D)
    be1r, be2r = be1.reshape(1, D), be2.reshape(1, D)

    deg_kernel, scatter_kernel = _sc_kernels()
    cnt = deg_kernel(dst_p, zdeg)
    h1 = _mm1(cnt, x_p, W1)
    s2 = scatter_kernel(h1, src_p, dst_p, zrows)
    h2 = _mid(cnt, s2, h1, b1r, g1r, be1r, W2)
    s2 = scatter_kernel(h2, src_p, dst_p, zrows)
    h3 = _mid(cnt, s2, h2, b2r, g2r, be2r, W3)
    s2 = scatter_kernel(h3, src_p, dst_p, zrows)
    out = _final(cnt, s2, h3, b3r)
    return out[:N]
